# baseline (speedup 1.0000x reference)
import jax
import jax.numpy as jnp
from jax.experimental import pallas as pl
from jax.experimental.pallas import tpu as pltpu
from jax.experimental.pallas import tpu_sc as plsc

_RB = 32
_CB = 512
_LANES = 16


def kernel(x, pe):
    batch, seq, d_model = x.shape
    rows = batch * seq
    x2d = x.reshape(rows, d_model)
    n_r = rows // _RB
    n_c = d_model // _CB
    n_pe = seq // _RB

    mesh = plsc.VectorSubcoreMesh(core_axis_name="core", subcore_axis_name="subcore")

    @pl.kernel(out_type=jax.ShapeDtypeStruct((rows, d_model), x.dtype), mesh=mesh)
    def sc_add(x_hbm, pe_hbm, o_hbm):
        def body(x_vmem, pe_vmem, o_vmem):
            @pl.loop(0, _RB)
            def _(r):
                @pl.loop(0, _CB, step=_LANES)
                def _(c):
                    slc = (pl.ds(r, 1), pl.ds(c, _LANES))
                    o_vmem.at[*slc][...] = x_vmem.at[*slc][...] + pe_vmem.at[*slc][...]

        pltpu.emit_pipeline(
            body,
            grid=(n_r, n_c),
            in_specs=[
                pl.BlockSpec((_RB, _CB), index_map=lambda i, j: (i, j)),
                pl.BlockSpec((_RB, _CB), index_map=lambda i, j: (i % n_pe, j)),
            ],
            out_specs=[pl.BlockSpec((_RB, _CB), index_map=lambda i, j: (i, j))],
            core_axis_name=("core", "subcore"),
            dimension_semantics=(pltpu.PARALLEL, pltpu.PARALLEL),
        )(x_hbm, pe_hbm, o_hbm)

    out = sc_add(x2d, pe)
    return out.reshape(batch, seq, d_model)


# speedup vs baseline: 1.0828x; 1.0828x over previous
import jax
import jax.numpy as jnp
from jax.experimental import pallas as pl
from jax.experimental.pallas import tpu as pltpu
from jax.experimental.pallas import tpu_sc as plsc

_RB = 32
_CB = 512
_LANES = 16


def kernel(x, pe):
    batch, seq, d_model = x.shape
    rows = batch * seq
    x2d = x.reshape(rows, d_model)
    n_r = rows // _RB
    n_c = d_model // _CB
    n_pe = seq // _RB

    mesh = plsc.VectorSubcoreMesh(core_axis_name="core", subcore_axis_name="subcore")

    @pl.kernel(out_type=jax.ShapeDtypeStruct((rows, d_model), x.dtype), mesh=mesh)
    def sc_add(x_hbm, pe_hbm, o_hbm):
        unroll = 16

        def body(x_vmem, pe_vmem, o_vmem):
            @pl.loop(0, _RB)
            def _(r):
                @pl.loop(0, _CB, step=_LANES * unroll)
                def _(c):
                    for u in range(unroll):
                        slc = (pl.ds(r, 1), pl.ds(c + u * _LANES, _LANES))
                        o_vmem.at[*slc][...] = (
                            x_vmem.at[*slc][...] + pe_vmem.at[*slc][...]
                        )

        pltpu.emit_pipeline(
            body,
            grid=(n_r, n_c),
            in_specs=[
                pl.BlockSpec((_RB, _CB), index_map=lambda i, j: (i, j)),
                pl.BlockSpec((_RB, _CB), index_map=lambda i, j: (i % n_pe, j)),
            ],
            out_specs=[pl.BlockSpec((_RB, _CB), index_map=lambda i, j: (i, j))],
            core_axis_name=("core", "subcore"),
            dimension_semantics=(pltpu.PARALLEL, pltpu.PARALLEL),
        )(x_hbm, pe_hbm, o_hbm)

    out = sc_add(x2d, pe)
    return out.reshape(batch, seq, d_model)


# speedup vs baseline: 4.0789x; 3.7670x over previous
import jax
import jax.numpy as jnp
from jax.experimental import pallas as pl
from jax.experimental.pallas import tpu as pltpu

_S_BLK = 128


def _add_pe_kernel(x_ref, pe_ref, o_ref):
    o_ref[...] = x_ref[...] + pe_ref[...][None, :, :]


def kernel(x, pe):
    batch, seq, d_model = x.shape
    grid = (seq // _S_BLK,)
    return pl.pallas_call(
        _add_pe_kernel,
        grid=grid,
        in_specs=[
            pl.BlockSpec((batch, _S_BLK, d_model), lambda i: (0, i, 0)),
            pl.BlockSpec((_S_BLK, d_model), lambda i: (i, 0)),
        ],
        out_specs=pl.BlockSpec((batch, _S_BLK, d_model), lambda i: (0, i, 0)),
        out_shape=jax.ShapeDtypeStruct((batch, seq, d_model), x.dtype),
        compiler_params=pltpu.CompilerParams(
            dimension_semantics=("parallel",),
        ),
    )(x, pe)


# speedup vs baseline: 4.3589x; 1.0687x over previous
import jax
import jax.numpy as jnp
from jax.experimental import pallas as pl
from jax.experimental.pallas import tpu as pltpu

_S_BLK = 512


def _add_pe_kernel(x_ref, pe_ref, o_ref):
    o_ref[...] = x_ref[...] + pe_ref[...][None, :, :]


def kernel(x, pe):
    batch, seq, d_model = x.shape
    grid = (seq // _S_BLK,)
    return pl.pallas_call(
        _add_pe_kernel,
        grid=grid,
        in_specs=[
            pl.BlockSpec((batch, _S_BLK, d_model), lambda i: (0, i, 0)),
            pl.BlockSpec((_S_BLK, d_model), lambda i: (i, 0)),
        ],
        out_specs=pl.BlockSpec((batch, _S_BLK, d_model), lambda i: (0, i, 0)),
        out_shape=jax.ShapeDtypeStruct((batch, seq, d_model), x.dtype),
        compiler_params=pltpu.CompilerParams(
            dimension_semantics=("parallel",),
        ),
    )(x, pe)
